# initial kernel scaffold (unmeasured)
import functools

import jax
import jax.numpy as jnp
from jax import lax
from jax.experimental import pallas as pl
from jax.experimental.pallas import tpu as pltpu

N_DEV = 4
S_CHUNK = 512


def kernel(O, Wo):
    B, S, H, D = O.shape
    K = H * D
    N = Wo.shape[1]

    O2 = O.reshape(B, S, K).astype(jnp.bfloat16)
    Wo2 = Wo.astype(jnp.bfloat16)

    def body(o_ref, w_ref, out_ref, send_buf, recv_buf, send_sems, recv_sems):
        my = lax.axis_index("i")
        left = lax.rem(my + N_DEV - 1, N_DEV)
        right = lax.rem(my + 1, N_DEV)

        def partial(c):
            o = o_ref[:, pl.ds(c * S_CHUNK, S_CHUNK), :]
            return lax.dot_general(
                o, w_ref[:, :],
                dimension_numbers=(((2,), (0,)), ((), ())),
                preferred_element_type=jnp.float32,
            )

        c0 = lax.rem(my + N_DEV - 1, N_DEV)
        send_buf[0] = partial(c0).astype(jnp.bfloat16)

        barrier_sem = pltpu.get_barrier_semaphore()
        for nbr in [left, right]:
            pl.semaphore_signal(
                barrier_sem, inc=1,
                device_id=(nbr,), device_id_type=pl.DeviceIdType.MESH,
            )
        pl.semaphore_wait(barrier_sem, 2)

        for s in range(N_DEV - 1):
            rdma = pltpu.make_async_remote_copy(
                src_ref=send_buf.at[s],
                dst_ref=recv_buf.at[s],
                send_sem=send_sems.at[s],
                recv_sem=recv_sems.at[s],
                device_id=(right,),
                device_id_type=pl.DeviceIdType.MESH,
            )
            rdma.start()
            c = lax.rem(my + 2 * N_DEV - s - 2, N_DEV)
            p = partial(c)
            rdma.wait()
            total = p + recv_buf[s].astype(jnp.float32)
            if s < N_DEV - 2:
                send_buf[s + 1] = total.astype(jnp.bfloat16)
            else:
                out_ref[:, :, :] = total

    return pl.pallas_call(
        body,
        out_shape=jax.ShapeDtypeStruct((B, S_CHUNK, N), jnp.float32),
        in_specs=[
            pl.BlockSpec(memory_space=pltpu.VMEM),
            pl.BlockSpec(memory_space=pltpu.VMEM),
        ],
        out_specs=pl.BlockSpec(memory_space=pltpu.VMEM),
        scratch_shapes=[
            pltpu.VMEM((N_DEV - 1, B, S_CHUNK, N), jnp.bfloat16),
            pltpu.VMEM((N_DEV - 1, B, S_CHUNK, N), jnp.bfloat16),
            pltpu.SemaphoreType.DMA((N_DEV - 1,)),
            pltpu.SemaphoreType.DMA((N_DEV - 1,)),
        ],
        compiler_params=pltpu.CompilerParams(collective_id=0),
    )(O2, Wo2)


# baseline (device time: 362024 ns/iter reference)
import jax
import jax.numpy as jnp
from jax import lax
from jax.experimental import pallas as pl
from jax.experimental.pallas import tpu as pltpu

N_DEV = 4
S_CHUNK = 512
N_SUB = 2
SUB_ROWS = S_CHUNK // N_SUB
N_TILES = 4


def kernel(O, Wo):
    B, S, H, D = O.shape
    K = H * D
    N = Wo.shape[1]
    NT = N // N_TILES

    O2 = O.reshape(B, S, K).astype(jnp.bfloat16)
    Wo2 = Wo.astype(jnp.bfloat16)

    def body(o_ref, w_ref, out_ref, send_buf, recv_buf,
             send_sem, recv_sem, credit_sem):
        my = lax.axis_index("i")
        left = lax.rem(my + N_DEV - 1, N_DEV)
        right = lax.rem(my + 1, N_DEV)

        def accumulate(c, row0, dst_ref, dst_row0, add_recv, to_bf16):
            o = o_ref[:, pl.ds(c * S_CHUNK + row0, SUB_ROWS), :]
            for nt in range(N_TILES):
                sl = slice(nt * NT, (nt + 1) * NT)
                p = lax.dot_general(
                    o, w_ref[:, sl],
                    dimension_numbers=(((2,), (0,)), ((), ())),
                    preferred_element_type=jnp.float32,
                )
                if add_recv:
                    p = p + recv_buf[:, :, sl].astype(jnp.float32)
                dst_ref[:, pl.ds(dst_row0, SUB_ROWS), sl] = (
                    p.astype(jnp.bfloat16) if to_bf16 else p
                )

        barrier_sem = pltpu.get_barrier_semaphore()
        for nbr in [left, right]:
            pl.semaphore_signal(
                barrier_sem, inc=1,
                device_id=(nbr,), device_id_type=pl.DeviceIdType.MESH,
            )
        pl.semaphore_wait(barrier_sem, 2)

        c0 = lax.rem(my + N_DEV - 1, N_DEV)

        for sub in range(N_SUB):
            row0 = sub * SUB_ROWS
            accumulate(c0, row0, send_buf, 0, add_recv=False, to_bf16=True)
            for s in range(N_DEV - 1):
                if sub + s > 0:
                    pl.semaphore_wait(credit_sem, 1)
                rdma = pltpu.make_async_remote_copy(
                    src_ref=send_buf,
                    dst_ref=recv_buf,
                    send_sem=send_sem,
                    recv_sem=recv_sem,
                    device_id=(right,),
                    device_id_type=pl.DeviceIdType.MESH,
                )
                rdma.start()
                rdma.wait()
                c = lax.rem(my + 2 * N_DEV - s - 2, N_DEV)
                last = s == N_DEV - 2
                accumulate(
                    c, row0,
                    out_ref if last else send_buf,
                    row0 if last else 0,
                    add_recv=True,
                    to_bf16=not last,
                )
                if not (sub == N_SUB - 1 and last):
                    pl.semaphore_signal(
                        credit_sem, inc=1,
                        device_id=(left,), device_id_type=pl.DeviceIdType.MESH,
                    )

    return pl.pallas_call(
        body,
        out_shape=jax.ShapeDtypeStruct((B, S_CHUNK, N), jnp.float32),
        in_specs=[
            pl.BlockSpec(memory_space=pltpu.VMEM),
            pl.BlockSpec(memory_space=pltpu.VMEM),
        ],
        out_specs=pl.BlockSpec(memory_space=pltpu.VMEM),
        scratch_shapes=[
            pltpu.VMEM((B, SUB_ROWS, N), jnp.bfloat16),
            pltpu.VMEM((B, SUB_ROWS, N), jnp.bfloat16),
            pltpu.SemaphoreType.DMA,
            pltpu.SemaphoreType.DMA,
            pltpu.SemaphoreType.REGULAR,
        ],
        compiler_params=pltpu.CompilerParams(collective_id=0),
    )(O2, Wo2)


# device time: 319018 ns/iter; 1.1348x vs baseline; 1.1348x over previous
import jax
import jax.numpy as jnp
from jax import lax
from jax.experimental import pallas as pl
from jax.experimental.pallas import tpu as pltpu

N_DEV = 4
S_CHUNK = 512
N_SUB = 2
SUB_ROWS = S_CHUNK // N_SUB
N_TILES = 4


def kernel(O, Wo):
    B, S, H, D = O.shape
    K = H * D
    N = Wo.shape[1]
    NT = N // N_TILES

    O2 = O.reshape(B, S, K).astype(jnp.bfloat16)
    Wo2 = Wo.astype(jnp.bfloat16)

    def body(o_ref, w_ref, out_ref, send_buf, recv_buf,
             send_sems, recv_sems, credit_sems):
        my = lax.axis_index("i")
        left = lax.rem(my + N_DEV - 1, N_DEV)
        right = lax.rem(my + 1, N_DEV)

        def rdma_for(nt):
            return pltpu.make_async_remote_copy(
                src_ref=send_buf.at[nt],
                dst_ref=recv_buf.at[nt],
                send_sem=send_sems.at[nt],
                recv_sem=recv_sems.at[nt],
                device_id=(right,),
                device_id_type=pl.DeviceIdType.MESH,
            )

        def dot_tile(o_c, nt):
            return lax.dot_general(
                o_c, w_ref[:, nt * NT:(nt + 1) * NT],
                dimension_numbers=(((2,), (0,)), ((), ())),
                preferred_element_type=jnp.float32,
            )

        barrier_sem = pltpu.get_barrier_semaphore()
        for nbr in [left, right]:
            pl.semaphore_signal(
                barrier_sem, inc=1,
                device_id=(nbr,), device_id_type=pl.DeviceIdType.MESH,
            )
        pl.semaphore_wait(barrier_sem, 2)

        c0 = lax.rem(my + N_DEV - 1, N_DEV)

        for sub in range(N_SUB):
            row0 = sub * SUB_ROWS

            o_c = o_ref[:, pl.ds(c0 * S_CHUNK + row0, SUB_ROWS), :]
            for nt in range(N_TILES):
                p = dot_tile(o_c, nt)
                if sub > 0:
                    pl.semaphore_wait(credit_sems.at[nt], 1)
                    rdma_for(nt).wait_send()
                send_buf[nt] = p.astype(jnp.bfloat16)
                rdma_for(nt).start()

            for s in range(N_DEV - 1):
                c = lax.rem(my + 2 * N_DEV - s - 2, N_DEV)
                o_c = o_ref[:, pl.ds(c * S_CHUNK + row0, SUB_ROWS), :]
                last = s == N_DEV - 2
                for nt in range(N_TILES):
                    p = dot_tile(o_c, nt)
                    rdma_for(nt).wait_recv()
                    tot = p + recv_buf[nt].astype(jnp.float32)
                    if not (sub == N_SUB - 1 and last):
                        pl.semaphore_signal(
                            credit_sems.at[nt], inc=1,
                            device_id=(left,),
                            device_id_type=pl.DeviceIdType.MESH,
                        )
                    if last:
                        out_ref[:, pl.ds(row0, SUB_ROWS),
                                nt * NT:(nt + 1) * NT] = tot
                    else:
                        pl.semaphore_wait(credit_sems.at[nt], 1)
                        rdma_for(nt).wait_send()
                        send_buf[nt] = tot.astype(jnp.bfloat16)
                        rdma_for(nt).start()

        for nt in range(N_TILES):
            rdma_for(nt).wait_send()

    return pl.pallas_call(
        body,
        out_shape=jax.ShapeDtypeStruct((B, S_CHUNK, N), jnp.float32),
        in_specs=[
            pl.BlockSpec(memory_space=pltpu.VMEM),
            pl.BlockSpec(memory_space=pltpu.VMEM),
        ],
        out_specs=pl.BlockSpec(memory_space=pltpu.VMEM),
        scratch_shapes=[
            pltpu.VMEM((N_TILES, B, SUB_ROWS, NT), jnp.bfloat16),
            pltpu.VMEM((N_TILES, B, SUB_ROWS, NT), jnp.bfloat16),
            pltpu.SemaphoreType.DMA((N_TILES,)),
            pltpu.SemaphoreType.DMA((N_TILES,)),
            pltpu.SemaphoreType.REGULAR((N_TILES,)),
        ],
        compiler_params=pltpu.CompilerParams(collective_id=0),
    )(O2, Wo2)


# device time: 310359 ns/iter; 1.1665x vs baseline; 1.0279x over previous
import jax
import jax.numpy as jnp
from jax import lax
from jax.experimental import pallas as pl
from jax.experimental.pallas import tpu as pltpu

N_DEV = 4
S_CHUNK = 512
N_SUB = 2
SUB_ROWS = S_CHUNK // N_SUB
N_TILES = 4


def kernel(O, Wo):
    B, S, H, D = O.shape
    K = H * D
    N = Wo.shape[1]
    NT = N // N_TILES

    O2 = O.reshape(B, S, K).astype(jnp.bfloat16)
    Wo2 = Wo.astype(jnp.bfloat16)

    def body(o_ref, w_ref, out_ref, send_buf, recv_buf, out_stage,
             send_sems, recv_sems, copy_sems, credit_sems):
        my = lax.axis_index("i")
        left = lax.rem(my + N_DEV - 1, N_DEV)
        right = lax.rem(my + 1, N_DEV)

        def rdma_for(nt, t):
            return pltpu.make_async_remote_copy(
                src_ref=send_buf.at[nt],
                dst_ref=recv_buf.at[nt, t % 2],
                send_sem=send_sems.at[nt],
                recv_sem=recv_sems.at[nt, t % 2],
                device_id=(right,),
                device_id_type=pl.DeviceIdType.MESH,
            )

        def dot_tile(o_c, nt):
            return lax.dot_general(
                o_c, w_ref[:, nt * NT:(nt + 1) * NT],
                dimension_numbers=(((2,), (0,)), ((), ())),
                preferred_element_type=jnp.float32,
            )

        def load_o(c, row0):
            return o_ref[:, pl.ds(c * S_CHUNK + row0, SUB_ROWS), :]

        def signal_credit(nt):
            pl.semaphore_signal(
                credit_sems.at[nt], inc=1,
                device_id=(left,), device_id_type=pl.DeviceIdType.MESH,
            )

        store_count = [0]

        def out_copy(row0, nt, tot):
            k = store_count[0]
            store_count[0] += 1
            slot = k % 2
            dst = out_ref.at[:, pl.ds(row0, SUB_ROWS),
                             pl.ds(nt * NT, NT)]
            cp = pltpu.make_async_copy(out_stage.at[slot], dst,
                                       copy_sems.at[slot])
            if k >= 2:
                cp.wait()
            out_stage[slot] = tot
            cp.start()

        def seed(sub, t):
            c0 = lax.rem(my + N_DEV - 1, N_DEV)
            o_c = load_o(c0, sub * SUB_ROWS)
            for nt in range(N_TILES):
                p = dot_tile(o_c, nt)
                if t >= 2:
                    pl.semaphore_wait(credit_sems.at[nt], 1)
                if t >= 1:
                    rdma_for(nt, t - 1).wait_send()
                send_buf[nt] = p.astype(jnp.bfloat16)
                rdma_for(nt, t).start()

        def hop(sub, s, t, forward):
            row0 = sub * SUB_ROWS
            c = lax.rem(my + 2 * N_DEV - s - 2, N_DEV)
            o_c = load_o(c, row0)
            for nt in range(N_TILES):
                p = dot_tile(o_c, nt)
                rdma_for(nt, t).wait_recv()
                tot = p + recv_buf[nt, t % 2].astype(jnp.float32)
                if t + 2 <= 2 * (N_DEV - 1) - 1:
                    signal_credit(nt)
                if forward:
                    if t + 1 >= 2:
                        pl.semaphore_wait(credit_sems.at[nt], 1)
                    rdma_for(nt, t).wait_send()
                    send_buf[nt] = tot.astype(jnp.bfloat16)
                    rdma_for(nt, t + 1).start()
                else:
                    out_copy(row0, nt, tot)

        barrier_sem = pltpu.get_barrier_semaphore()
        for nbr in [left, right]:
            pl.semaphore_signal(
                barrier_sem, inc=1,
                device_id=(nbr,), device_id_type=pl.DeviceIdType.MESH,
            )
        pl.semaphore_wait(barrier_sem, 2)

        seed(0, 0)
        hop(0, 0, 0, forward=True)
        hop(0, 1, 1, forward=True)
        seed(1, 3)
        hop(0, 2, 2, forward=False)
        hop(1, 0, 3, forward=True)
        hop(1, 1, 4, forward=True)
        hop(1, 2, 5, forward=False)

        for nt in range(N_TILES):
            rdma_for(nt, 5).wait_send()
        for slot in range(2):
            pltpu.make_async_copy(
                out_stage.at[slot],
                out_ref.at[:, pl.ds(0, SUB_ROWS), pl.ds(slot * NT, NT)],
                copy_sems.at[slot],
            ).wait()

    return pl.pallas_call(
        body,
        out_shape=jax.ShapeDtypeStruct((B, S_CHUNK, N), jnp.float32),
        in_specs=[
            pl.BlockSpec(memory_space=pltpu.VMEM),
            pl.BlockSpec(memory_space=pltpu.VMEM),
        ],
        out_specs=pl.BlockSpec(memory_space=pl.ANY),
        scratch_shapes=[
            pltpu.VMEM((N_TILES, B, SUB_ROWS, NT), jnp.bfloat16),
            pltpu.VMEM((N_TILES, 2, B, SUB_ROWS, NT), jnp.bfloat16),
            pltpu.VMEM((2, B, SUB_ROWS, NT), jnp.float32),
            pltpu.SemaphoreType.DMA((N_TILES,)),
            pltpu.SemaphoreType.DMA((N_TILES, 2)),
            pltpu.SemaphoreType.DMA((2,)),
            pltpu.SemaphoreType.REGULAR((N_TILES,)),
        ],
        compiler_params=pltpu.CompilerParams(collective_id=0),
    )(O2, Wo2)
